# baseline (speedup 1.0000x reference)
import jax
import jax.numpy as jnp
from jax import lax
from jax.experimental import pallas as pl
from jax.experimental.pallas import tpu as pltpu
from jax.experimental.pallas import tpu_sc as plsc

ROWS = 16384
COLS = 256
NC = 2
NS = 16
L = 16
NW = NC * NS
RPW = ROWS // NW
CHUNK = 64
NCHUNK = RPW // CHUNK
NIB = 4
NOB = 2
GROUPS = COLS // L


def _permute_body(x_hbm, perm_hbm, out_hbm, perm_v, *bufs):
    xin = list(bufs[0:NIB])
    xout = list(bufs[NIB:NIB + NOB])
    isem = list(bufs[NIB + NOB:2 * NIB + NOB])
    osem = list(bufs[2 * NIB + NOB:2 * NIB + 2 * NOB])

    wid = lax.axis_index("s") * NC + lax.axis_index("c")
    row_base = wid * RPW

    perm_cp = pltpu.make_async_copy(perm_hbm, perm_v, osem[0])

    def in_copy(c, b):
        r0 = row_base + c * CHUNK
        return pltpu.make_async_copy(
            x_hbm.at[pl.ds(r0, CHUNK), :], xin[b], isem[b]
        )

    def out_copy(c, b):
        r0 = row_base + c * CHUNK
        return pltpu.make_async_copy(
            xout[b], out_hbm.at[pl.ds(r0, CHUNK), :], osem[b]
        )

    def compute(bi, bo):
        src = xin[bi]
        dst = xout[bo]

        @plsc.parallel_loop(0, CHUNK, unroll=4)
        def do_row(r):
            rvec = jnp.full((L,), r, dtype=jnp.int32)
            for g in range(GROUPS):
                dst[r, pl.ds(g * L, L)] = plsc.load_gather(
                    src, [rvec, idx0[g]])

    perm_cp.start()
    for b in range(NIB):
        in_copy(b, b).start()
    perm_cp.wait()
    idx0 = [perm_v[pl.ds(g * L, L)] for g in range(GROUPS)]

    def ring_body(i, _):
        for k in range(NIB):
            c = i * NIB + k
            bi = k
            bo = k % NOB
            in_copy(c, bi).wait()

            @pl.when(jnp.logical_and(c >= 2, c + 2 < NCHUNK))
            def _():
                in_copy(c + 2, (k + 2) % NIB).start()

            @pl.when(c >= NOB)
            def _():
                out_copy(c - NOB, bo).wait()

            compute(bi, bo)
            out_copy(c, bo).start()
        return 0

    lax.fori_loop(0, NCHUNK // NIB, ring_body, 0, unroll=False)
    for c in range(NCHUNK - NOB, NCHUNK):
        out_copy(c, c % NOB).wait()


@jax.jit
def kernel(x, permutation):
    mesh = plsc.VectorSubcoreMesh(core_axis_name="c", subcore_axis_name="s")
    run = pl.kernel(
        _permute_body,
        mesh=mesh,
        out_type=jax.ShapeDtypeStruct((ROWS, COLS), jnp.float32),
        compiler_params=pltpu.CompilerParams(needs_layout_passes=False),
        scratch_types=(
            [pltpu.VMEM((COLS,), jnp.int32)]
            + [pltpu.VMEM((CHUNK, COLS), jnp.float32)] * (NIB + NOB)
            + [pltpu.SemaphoreType.DMA] * (NIB + NOB)
        ),
    )
    return run(x, permutation)


# speedup vs baseline: 1.0337x; 1.0337x over previous
import jax
import jax.numpy as jnp
from jax import lax
from jax.experimental import pallas as pl
from jax.experimental.pallas import tpu as pltpu
from jax.experimental.pallas import tpu_sc as plsc

ROWS = 16384
COLS = 256
NC = 2
NS = 16
L = 16
NW = NC * NS
RPW = ROWS // NW
CHUNK = 64
NCHUNK = RPW // CHUNK
NIB = 4
NOB = 2
GROUPS = COLS // L


def _permute_body(x_hbm, perm_hbm, out_hbm, perm_v, *bufs):
    xin = list(bufs[0:NIB])
    xout = list(bufs[NIB:NIB + NOB])
    isem = list(bufs[NIB + NOB:2 * NIB + NOB])
    osem = list(bufs[2 * NIB + NOB:2 * NIB + 2 * NOB])

    wid = lax.axis_index("s") * NC + lax.axis_index("c")
    row_base = wid * RPW

    perm_cp = pltpu.make_async_copy(perm_hbm, perm_v, osem[0])

    def in_copy(c, b):
        r0 = row_base + c * CHUNK
        return pltpu.make_async_copy(
            x_hbm.at[pl.ds(r0, CHUNK), :], xin[b], isem[b]
        )

    def out_copy(c, b):
        r0 = row_base + c * CHUNK
        return pltpu.make_async_copy(
            xout[b], out_hbm.at[pl.ds(r0, CHUNK), :], osem[b]
        )

    def compute(bi, bo):
        src = xin[bi]
        dst = xout[bo]

        @plsc.parallel_loop(0, CHUNK, unroll=2)
        def do_row(r):
            rvec = jnp.full((L,), r, dtype=jnp.int32)
            for g in range(GROUPS):
                dst[r, pl.ds(g * L, L)] = plsc.load_gather(
                    src, [rvec, idx0[g]])

    perm_cp.start()
    for b in range(NIB):
        in_copy(b, b).start()
    perm_cp.wait()
    idx0 = [perm_v[pl.ds(g * L, L)] for g in range(GROUPS)]

    def ring_body(i, _):
        for k in range(NIB):
            c = i * NIB + k
            bi = k
            bo = k % NOB
            in_copy(c, bi).wait()

            @pl.when(jnp.logical_and(c >= 2, c + 2 < NCHUNK))
            def _():
                in_copy(c + 2, (k + 2) % NIB).start()

            @pl.when(c >= NOB)
            def _():
                out_copy(c - NOB, bo).wait()

            compute(bi, bo)
            out_copy(c, bo).start()
        return 0

    lax.fori_loop(0, NCHUNK // NIB, ring_body, 0, unroll=False)
    for c in range(NCHUNK - NOB, NCHUNK):
        out_copy(c, c % NOB).wait()


@jax.jit
def kernel(x, permutation):
    mesh = plsc.VectorSubcoreMesh(core_axis_name="c", subcore_axis_name="s")
    run = pl.kernel(
        _permute_body,
        mesh=mesh,
        out_type=jax.ShapeDtypeStruct((ROWS, COLS), jnp.float32),
        compiler_params=pltpu.CompilerParams(needs_layout_passes=False),
        scratch_types=(
            [pltpu.VMEM((COLS,), jnp.int32)]
            + [pltpu.VMEM((CHUNK, COLS), jnp.float32)] * (NIB + NOB)
            + [pltpu.SemaphoreType.DMA] * (NIB + NOB)
        ),
    )
    return run(x, permutation)


# speedup vs baseline: 1.0437x; 1.0097x over previous
import jax
import jax.numpy as jnp
from jax import lax
from jax.experimental import pallas as pl
from jax.experimental.pallas import tpu as pltpu
from jax.experimental.pallas import tpu_sc as plsc

ROWS = 16384
COLS = 256
NC = 2
NS = 16
L = 16
NW = NC * NS
RPW = ROWS // NW
CHUNK = 64
NCHUNK = RPW // CHUNK
NIB = 4
NOB = 2
GROUPS = COLS // L


def _permute_body(x_hbm, perm_hbm, out_hbm, perm_v, *bufs):
    xin = list(bufs[0:NIB])
    xout = list(bufs[NIB:NIB + NOB])
    isem = list(bufs[NIB + NOB:2 * NIB + NOB])
    osem = list(bufs[2 * NIB + NOB:2 * NIB + 2 * NOB])

    wid = lax.axis_index("s") * NC + lax.axis_index("c")
    row_base = wid * RPW

    pltpu.sync_copy(perm_hbm, perm_v)
    idx0 = [perm_v[pl.ds(g * L, L)] for g in range(GROUPS)]

    def in_copy(c, b):
        r0 = row_base + c * CHUNK
        return pltpu.make_async_copy(
            x_hbm.at[pl.ds(r0, CHUNK), :], xin[b], isem[b]
        )

    def out_copy(c, b):
        r0 = row_base + c * CHUNK
        return pltpu.make_async_copy(
            xout[b], out_hbm.at[pl.ds(r0, CHUNK), :], osem[b]
        )

    def compute(bi, bo):
        src = xin[bi]
        dst = xout[bo]

        @plsc.parallel_loop(0, CHUNK, unroll=2)
        def do_row(r):
            rvec = jnp.full((L,), r, dtype=jnp.int32)
            for g in range(GROUPS):
                dst[r, pl.ds(g * L, L)] = plsc.load_gather(
                    src, [rvec, idx0[g]])

    for b in range(NIB):
        in_copy(b, b).start()

    def ring_body(i, _):
        for k in range(NIB):
            c = i * NIB + k
            bi = k
            bo = k % NOB
            in_copy(c, bi).wait()

            @pl.when(c >= NOB)
            def _():
                out_copy(c - NOB, bo).wait()

            compute(bi, bo)

            @pl.when(c + NIB < NCHUNK)
            def _():
                in_copy(c + NIB, bi).start()

            out_copy(c, bo).start()
        return 0

    lax.fori_loop(0, NCHUNK // NIB, ring_body, 0, unroll=False)
    for c in range(NCHUNK - NOB, NCHUNK):
        out_copy(c, c % NOB).wait()


@jax.jit
def kernel(x, permutation):
    mesh = plsc.VectorSubcoreMesh(core_axis_name="c", subcore_axis_name="s")
    run = pl.kernel(
        _permute_body,
        mesh=mesh,
        out_type=jax.ShapeDtypeStruct((ROWS, COLS), jnp.float32),
        compiler_params=pltpu.CompilerParams(needs_layout_passes=False),
        scratch_types=(
            [pltpu.VMEM((COLS,), jnp.int32)]
            + [pltpu.VMEM((CHUNK, COLS), jnp.float32)] * (NIB + NOB)
            + [pltpu.SemaphoreType.DMA] * (NIB + NOB)
        ),
    )
    return run(x, permutation)
